# Initial kernel scaffold; baseline (speedup 1.0000x reference)
import jax
import jax.numpy as jnp
from jax.experimental import pallas as pl


def kernel(feats, proj_matrices, vol_origin_partial):
    raise NotImplementedError("write your pallas kernel here")


# speedup vs baseline: 13.8559x; 13.8559x over previous
import functools

import jax
import jax.numpy as jnp
from jax import lax
from jax.experimental import pallas as pl
from jax.experimental.pallas import tpu as pltpu
from jax.experimental.pallas import tpu_sc as plsc

N_VIEWS = 9
C = 80
H = 60
W = 80
NVOX = 48 * 48 * 48
NW = 32
SLAB = NVOX // NW
NCH = SLAB // 128
VOXEL_SIZE = 0.04


def _stage0_body(proj_ref, origin_ref, base_ref, wts_ref, cnt_ref, inv_ref,
                 imz_ref):
    w = pl.program_id(0)
    row = lax.broadcasted_iota(jnp.int32, (NCH, 128), 0)
    lane = lax.broadcasted_iota(jnp.int32, (NCH, 128), 1)
    nf = (SLAB * w + 128 * row + lane).astype(jnp.float32)
    fi = jnp.floor(nf / 2304.0)
    rem = nf - fi * 2304.0
    fj = jnp.floor(rem / 48.0)
    fk = rem - fj * 48.0
    ox = origin_ref[0, 0]
    oy = origin_ref[0, 1]
    oz = origin_ref[0, 2]
    wx = (4.0 * fi) * jnp.float32(VOXEL_SIZE) + ox
    wy = (4.0 * fj) * jnp.float32(VOXEL_SIZE) + oy
    wz = (4.0 * fk) * jnp.float32(VOXEL_SIZE) + oz

    cnt = jnp.zeros((NCH, 128), jnp.float32)
    zsum = jnp.zeros((NCH, 128), jnp.float32)
    for v in range(N_VIEWS):
        p00 = proj_ref[v, 0, 0, 0]
        p01 = proj_ref[v, 0, 0, 1]
        p02 = proj_ref[v, 0, 0, 2]
        p03 = proj_ref[v, 0, 0, 3]
        p10 = proj_ref[v, 0, 1, 0]
        p11 = proj_ref[v, 0, 1, 1]
        p12 = proj_ref[v, 0, 1, 2]
        p13 = proj_ref[v, 0, 1, 3]
        p20 = proj_ref[v, 0, 2, 0]
        p21 = proj_ref[v, 0, 2, 1]
        p22 = proj_ref[v, 0, 2, 2]
        p23 = proj_ref[v, 0, 2, 3]
        q0 = p00 * wx + p01 * wy + p02 * wz + p03
        q1 = p10 * wx + p11 * wy + p12 * wz + p13
        q2 = p20 * wx + p21 * wy + p22 * wz + p23
        den = q2 + jnp.float32(1e-8)
        im_x = q0 / den
        im_y = q1 / den
        gx = 2.0 * im_x / (W - 1) - 1.0
        gy = 2.0 * im_y / (H - 1) - 1.0
        mask = (jnp.abs(gx) <= 1.0) & (jnp.abs(gy) <= 1.0) & (q2 > 0)
        ix = (gx + 1.0) * 0.5 * (W - 1)
        iy = (gy + 1.0) * 0.5 * (H - 1)
        x0 = jnp.clip(jnp.floor(ix), 0.0, W - 2.0)
        y0 = jnp.clip(jnp.floor(iy), 0.0, H - 2.0)
        wx1 = ix - x0
        wx0 = 1.0 - wx1
        wy1 = iy - y0
        wy0 = 1.0 - wy1
        base = jnp.where(mask, y0 * jnp.float32(W) + x0, 0.0)
        base_ref[0, :, v, :] = base.astype(jnp.int32)
        wts_ref[0, :, v, 0, :] = jnp.where(mask, wx0 * wy0, 0.0)
        wts_ref[0, :, v, 1, :] = jnp.where(mask, wx1 * wy0, 0.0)
        wts_ref[0, :, v, 2, :] = jnp.where(mask, wx0 * wy1, 0.0)
        wts_ref[0, :, v, 3, :] = jnp.where(mask, wx1 * wy1, 0.0)
        cnt = cnt + mask.astype(jnp.float32)
        zsum = zsum + jnp.where(mask, q2, 0.0)

    msafe = jnp.where(cnt == 0.0, 1.0, cnt)
    cnt_ref[0] = cnt
    inv_ref[0] = 1.0 / msafe
    imz_ref[0] = zsum / msafe


def _stage0(proj, origin):
    return pl.pallas_call(
        _stage0_body,
        grid=(NW,),
        in_specs=[
            pl.BlockSpec(memory_space=pltpu.SMEM),
            pl.BlockSpec(memory_space=pltpu.SMEM),
        ],
        out_specs=[
            pl.BlockSpec((1, NCH, N_VIEWS, 128), lambda w: (w, 0, 0, 0)),
            pl.BlockSpec((1, NCH, N_VIEWS, 4, 128), lambda w: (w, 0, 0, 0, 0)),
            pl.BlockSpec((1, NCH, 128), lambda w: (w, 0, 0)),
            pl.BlockSpec((1, NCH, 128), lambda w: (w, 0, 0)),
            pl.BlockSpec((1, NCH, 128), lambda w: (w, 0, 0)),
        ],
        out_shape=[
            jax.ShapeDtypeStruct((NW, NCH, N_VIEWS, 128), jnp.int32),
            jax.ShapeDtypeStruct((NW, NCH, N_VIEWS, 4, 128), jnp.float32),
            jax.ShapeDtypeStruct((NW, NCH, 128), jnp.float32),
            jax.ShapeDtypeStruct((NW, NCH, 128), jnp.float32),
            jax.ShapeDtypeStruct((NW, NCH, 128), jnp.float32),
        ],
    )(proj, origin)


def _stage1_body(tables, base_hbm, wts_hbm, inv_hbm, out_hbm,
                 w4_v, base_v, inv_v, idx4, rows, acc, sem):
    wid = lax.axis_index("s") * 2 + lax.axis_index("c")

    def view_work(v, init):
        handles = [
            pltpu.async_copy(tables.at[v].at[idx4.at[v, ci]],
                             rows.at[ci], sem)
            for ci in range(4)
        ]
        for h in handles:
            h.wait()

        def group_body(g, _):
            w16 = [w4_v[v, ci, pl.ds(g * 16, 16)] for ci in range(4)]
            for jj in range(16):
                j = g * 16 + jj
                w0 = w16[0][jj]
                w1 = w16[1][jj]
                w2 = w16[2][jj]
                w3 = w16[3][jj]
                for k in range(5):
                    sl = pl.ds(16 * k, 16)
                    s = (w0 * rows[0, j, sl] + w1 * rows[1, j, sl]
                         + w2 * rows[2, j, sl] + w3 * rows[3, j, sl])
                    if init:
                        acc[j, sl] = s
                    else:
                        acc[j, sl] = acc[j, sl] + s
            return 0

        lax.fori_loop(0, 8, group_body, 0)

    def chunk_body(r, _):
        pltpu.sync_copy(wts_hbm.at[wid, r], w4_v)
        pltpu.sync_copy(base_hbm.at[wid, r], base_v)
        pltpu.sync_copy(inv_hbm.at[wid, r], inv_v)

        def idx_body(v, _):
            for ci, off in enumerate((0, 1, W, W + 1)):
                for s in range(8):
                    sl = pl.ds(16 * s, 16)
                    idx4[v, ci, sl] = base_v[v, sl] + off
            return 0

        lax.fori_loop(0, N_VIEWS, idx_body, 0)

        view_work(0, True)

        def view_body(v, _):
            view_work(v, False)
            return 0

        lax.fori_loop(1, N_VIEWS, view_body, 0)

        def scale_body(g, _):
            iv16 = inv_v[pl.ds(g * 16, 16)]
            for jj in range(16):
                j = g * 16 + jj
                iv = iv16[jj]
                for k in range(5):
                    sl = pl.ds(16 * k, 16)
                    acc[j, sl] = acc[j, sl] * iv
            return 0

        lax.fori_loop(0, 8, scale_body, 0)
        pltpu.sync_copy(acc, out_hbm.at[wid, pl.ds(r * 128, 128)])
        return 0

    lax.fori_loop(0, NCH, chunk_body, 0)


def _stage1(tables, base, wts, inv):
    mesh = plsc.VectorSubcoreMesh(core_axis_name="c", subcore_axis_name="s")
    f = functools.partial(
        pl.kernel,
        out_type=jax.ShapeDtypeStruct((NW, SLAB, C), jnp.float32),
        mesh=mesh,
        compiler_params=pltpu.CompilerParams(use_tc_tiling_on_sc=False),
        scratch_types=[
            pltpu.VMEM((N_VIEWS, 4, 128), jnp.float32),
            pltpu.VMEM((N_VIEWS, 128), jnp.int32),
            pltpu.VMEM((128,), jnp.float32),
            pltpu.VMEM((N_VIEWS, 4, 128), jnp.int32),
            pltpu.VMEM((4, 128, C), jnp.float32),
            pltpu.VMEM((128, C), jnp.float32),
            pltpu.SemaphoreType.DMA,
        ],
    )(_stage1_body)
    return f(tables, base, wts, inv)


def _stage2_body(imz_ref, imz_col_ref, feat_ref, vol_ref):
    x = imz_ref[...].reshape(NW * NCH, 128)
    pos = x > 0.0
    posf = pos.astype(jnp.float32)
    npos = jnp.sum(posf)
    mean = jnp.where(npos > 0.0,
                     jnp.sum(jnp.where(pos, x, 0.0)) / jnp.maximum(npos, 1.0),
                     0.0)
    std = jnp.sqrt(jnp.sum(jnp.where(pos, (x - mean) ** 2, 0.0))) + 1e-5
    xc = imz_col_ref[...].reshape(SLAB, 1)
    posc = xc > 0.0
    zn = jnp.where(posc, (xc - mean) / std, 0.0)
    vol_ref[:, 0:C] = feat_ref[0]
    vol_ref[:, C:C + 1] = zn


def _stage2(imz, imz_col, feat):
    return pl.pallas_call(
        _stage2_body,
        grid=(NW,),
        in_specs=[
            pl.BlockSpec((NW, NCH, 128), lambda w: (0, 0, 0)),
            pl.BlockSpec((1, SLAB, 1), lambda w: (w, 0, 0)),
            pl.BlockSpec((1, SLAB, C), lambda w: (w, 0, 0)),
        ],
        out_specs=pl.BlockSpec((SLAB, C + 1), lambda w: (w, 0)),
        out_shape=jax.ShapeDtypeStruct((NVOX, C + 1), jnp.float32),
    )(imz, imz_col, feat)


def kernel(feats, proj_matrices, vol_origin_partial):
    tables = jnp.transpose(feats[:, 0], (0, 2, 3, 1)).reshape(N_VIEWS, H * W, C)
    base, wts, cnt, inv, imz = _stage0(proj_matrices, vol_origin_partial)
    feat = _stage1(tables, base, wts, inv)
    volume = _stage2(imz, imz.reshape(NW, SLAB, 1), feat)
    return volume, cnt.reshape(NVOX)


# speedup vs baseline: 23.2556x; 1.6784x over previous
import functools

import jax
import jax.numpy as jnp
from jax import lax
from jax.experimental import pallas as pl
from jax.experimental.pallas import tpu as pltpu
from jax.experimental.pallas import tpu_sc as plsc

N_VIEWS = 9
C = 80
H = 60
W = 80
NVOX = 48 * 48 * 48
NW = 32
SLAB = NVOX // NW
NCH = SLAB // 128
VOXEL_SIZE = 0.04
OFFS = (0, 1, W, W + 1)


def _stage0_body(pix_ref, idx_ref, wts_ref, cnt_ref, inv_ref, imz_ref):
    cnt = jnp.zeros((NW, 128), jnp.float32)
    zsum = jnp.zeros((NW, 128), jnp.float32)
    for v in range(N_VIEWS):
        im_x = pix_ref[0, v, 0]
        im_y = pix_ref[0, v, 1]
        q2 = pix_ref[0, v, 2]
        gx = 2.0 * im_x / (W - 1) - 1.0
        gy = 2.0 * im_y / (H - 1) - 1.0
        mask = (jnp.abs(gx) <= 1.0) & (jnp.abs(gy) <= 1.0) & (q2 > 0)
        ix = (gx + 1.0) * 0.5 * (W - 1)
        iy = (gy + 1.0) * 0.5 * (H - 1)
        x0 = jnp.clip(jnp.floor(ix), 0.0, W - 2.0)
        y0 = jnp.clip(jnp.floor(iy), 0.0, H - 2.0)
        wx1 = ix - x0
        wx0 = 1.0 - wx1
        wy1 = iy - y0
        wy0 = 1.0 - wy1
        base = jnp.where(mask, y0 * jnp.float32(W) + x0, 0.0).astype(jnp.int32)
        for ci, off in enumerate(OFFS):
            idx_ref[0, :, v, ci, :] = base + off
        wts_ref[0, :, v, 0, :] = jnp.where(mask, wx0 * wy0, 0.0)
        wts_ref[0, :, v, 1, :] = jnp.where(mask, wx1 * wy0, 0.0)
        wts_ref[0, :, v, 2, :] = jnp.where(mask, wx0 * wy1, 0.0)
        wts_ref[0, :, v, 3, :] = jnp.where(mask, wx1 * wy1, 0.0)
        cnt = cnt + mask.astype(jnp.float32)
        zsum = zsum + jnp.where(mask, q2, 0.0)

    msafe = jnp.where(cnt == 0.0, 1.0, cnt)
    cnt_ref[0, :, 0, :] = cnt
    inv_ref[0, :, 0, :] = 1.0 / msafe
    imz_ref[0, :, 0, :] = zsum / msafe


def _stage0(pix):
    return pl.pallas_call(
        _stage0_body,
        grid=(NCH,),
        in_specs=[
            pl.BlockSpec((1, N_VIEWS, 3, NW, 128), lambda r: (r, 0, 0, 0, 0)),
        ],
        out_specs=[
            pl.BlockSpec((1, NW, N_VIEWS, 4, 128), lambda r: (r, 0, 0, 0, 0)),
            pl.BlockSpec((1, NW, N_VIEWS, 4, 128), lambda r: (r, 0, 0, 0, 0)),
            pl.BlockSpec((1, NW, 1, 128), lambda r: (r, 0, 0, 0)),
            pl.BlockSpec((1, NW, 1, 128), lambda r: (r, 0, 0, 0)),
            pl.BlockSpec((1, NW, 1, 128), lambda r: (r, 0, 0, 0)),
        ],
        out_shape=[
            jax.ShapeDtypeStruct((NCH, NW, N_VIEWS, 4, 128), jnp.int32),
            jax.ShapeDtypeStruct((NCH, NW, N_VIEWS, 4, 128), jnp.float32),
            jax.ShapeDtypeStruct((NCH, NW, 1, 128), jnp.float32),
            jax.ShapeDtypeStruct((NCH, NW, 1, 128), jnp.float32),
            jax.ShapeDtypeStruct((NCH, NW, 1, 128), jnp.float32),
        ],
    )(pix)


def _stagez_body(imz_ref, zn_ref):
    x = imz_ref[...].reshape(NCH * NW, 128)
    pos = x > 0.0
    npos = jnp.sum(pos.astype(jnp.float32))
    mean = jnp.where(npos > 0.0,
                     jnp.sum(jnp.where(pos, x, 0.0)) / jnp.maximum(npos, 1.0),
                     0.0)
    std = jnp.sqrt(jnp.sum(jnp.where(pos, (x - mean) ** 2, 0.0))) + 1e-5
    zn = jnp.where(pos, (x - mean) / std, 0.0)
    zn_ref[...] = zn.reshape(NCH, NW, 1, 128)


def _stagez(imz):
    return pl.pallas_call(
        _stagez_body,
        out_shape=jax.ShapeDtypeStruct((NCH, NW, 1, 128), jnp.float32),
    )(imz)


def _stage1_body(tables, idx_hbm, wts_hbm, inv_hbm, zn_hbm, vol_hbm,
                 idx_s, w_s, inv_s, zn_s, rows, acc, sem_s, sem_a, sem_b):
    wid = lax.axis_index("s") * 2 + lax.axis_index("c")

    def fetch_slabs(r, p):
        pltpu.async_copy(idx_hbm.at[r, wid], idx_s.at[p], sem_s)
        pltpu.async_copy(wts_hbm.at[r, wid], w_s.at[p, :, :, pl.ds(0, 128)],
                         sem_s)
        pltpu.async_copy(inv_hbm.at[r, wid, 0], inv_s.at[p, pl.ds(0, 128)],
                         sem_s)
        pltpu.async_copy(zn_hbm.at[r, wid, 0], zn_s.at[p, pl.ds(0, 128)],
                         sem_s)

    def wait_slabs(p):
        pltpu.make_async_copy(idx_hbm.at[0, 0], idx_s.at[p], sem_s).wait()
        pltpu.make_async_copy(wts_hbm.at[0, 0],
                              w_s.at[p, :, :, pl.ds(0, 128)], sem_s).wait()
        pltpu.make_async_copy(inv_hbm.at[0, 0, 0],
                              inv_s.at[p, pl.ds(0, 128)], sem_s).wait()
        pltpu.make_async_copy(zn_hbm.at[0, 0, 0],
                              zn_s.at[p, pl.ds(0, 128)], sem_s).wait()

    fetch_slabs(0, 0)

    def chunk_body(r, _):
        p = lax.rem(r, 2)
        wait_slabs(p)

        def fire(v, sem):
            return [pltpu.async_copy(tables.at[v].at[idx_s.at[p, v, ci]],
                                     rows.at[v % 2, ci], sem)
                    for ci in range(4)]

        h = fire(0, sem_a)
        fetch_slabs(jnp.minimum(r + 1, NCH - 1), 1 - p)

        for v in range(N_VIEWS):
            hn = fire(v + 1, (sem_a, sem_b)[(v + 1) % 2]) if v < N_VIEWS - 1 else None
            for hh in h:
                hh.wait()

            def group_body(g, _, v=v):
                w16 = [w_s[p, v, ci, pl.ds(g * 8, 16)] for ci in range(4)]
                for jj in range(8):
                    j = g * 8 + jj
                    w0 = w16[0][jj]
                    w1 = w16[1][jj]
                    w2 = w16[2][jj]
                    w3 = w16[3][jj]
                    for k in range(5):
                        sl = pl.ds(16 * k, 16)
                        s = (w0 * rows[v % 2, 0, j, sl]
                             + w1 * rows[v % 2, 1, j, sl]
                             + w2 * rows[v % 2, 2, j, sl]
                             + w3 * rows[v % 2, 3, j, sl])
                        if v == 0:
                            acc[j, sl] = s
                        else:
                            acc[j, sl] = acc[j, sl] + s
                return 0

            lax.fori_loop(0, 16, group_body, 0)
            h = hn

        col80 = jnp.full((16,), C, jnp.int32)

        def scale_body(g, _):
            iv16 = inv_s[p, pl.ds(g * 8, 16)]
            for jj in range(8):
                j = g * 8 + jj
                iv = iv16[jj]
                for k in range(5):
                    sl = pl.ds(16 * k, 16)
                    acc[j, sl] = acc[j, sl] * iv
            return 0

        lax.fori_loop(0, 16, scale_body, 0)

        def zcol_body(g, _):
            z16 = zn_s[p, pl.ds(g * 16, 16)]
            rows16 = g * 16 + lax.iota(jnp.int32, 16)
            plsc.store_scatter(acc, [rows16, col80], z16)
            return 0

        lax.fori_loop(0, 8, zcol_body, 0)
        pltpu.sync_copy(acc, vol_hbm.at[pl.ds((r * NW + wid) * 128, 128)])
        return 0

    lax.fori_loop(0, NCH, chunk_body, 0)
    wait_slabs(NCH % 2)


def _stage1(tables, idx4, wts, inv, zn):
    mesh = plsc.VectorSubcoreMesh(core_axis_name="c", subcore_axis_name="s")
    f = functools.partial(
        pl.kernel,
        out_type=jax.ShapeDtypeStruct((NVOX, C + 1), jnp.float32),
        mesh=mesh,
        compiler_params=pltpu.CompilerParams(use_tc_tiling_on_sc=False,
                                             needs_layout_passes=False),
        scratch_types=[
            pltpu.VMEM((2, N_VIEWS, 4, 128), jnp.int32),
            pltpu.VMEM((2, N_VIEWS, 4, 144), jnp.float32),
            pltpu.VMEM((2, 144), jnp.float32),
            pltpu.VMEM((2, 144), jnp.float32),
            pltpu.VMEM((2, 4, 128, C), jnp.float32),
            pltpu.VMEM((128, C + 1), jnp.float32),
            pltpu.SemaphoreType.DMA,
            pltpu.SemaphoreType.DMA,
            pltpu.SemaphoreType.DMA,
        ],
    )(_stage1_body)
    return f(tables, idx4, wts, inv, zn)


def kernel(feats, proj_matrices, vol_origin_partial):
    tables = jnp.transpose(feats[:, 0], (0, 2, 3, 1)).reshape(N_VIEWS, H * W, C)
    interval = 4
    rngs = [jnp.arange(0, 48 * interval, interval, dtype=jnp.float32)
            for _ in range(3)]
    g = jnp.stack(jnp.meshgrid(rngs[0], rngs[1], rngs[2], indexing='ij'))
    coords_xyz = g.reshape(3, -1).T
    grid = coords_xyz * VOXEL_SIZE + vol_origin_partial[0][None, :]
    rs = jnp.concatenate([grid.T, jnp.ones((1, NVOX), grid.dtype)], axis=0)
    im_p = lax.map(lambda P: P @ rs, proj_matrices[:, 0])
    im_z = im_p[:, 2]
    im_x = im_p[:, 0] / (im_z + 1e-8)
    im_y = im_p[:, 1] / (im_z + 1e-8)
    pix = jnp.stack([im_x, im_y, im_z], axis=1)
    pix = pix.reshape(N_VIEWS, 3, NCH, NW, 128).transpose(2, 0, 1, 3, 4)

    idx4, wts, cnt, inv, imz = _stage0(pix)
    zn = _stagez(imz)
    volume = _stage1(tables, idx4, wts, inv, zn)
    return volume, cnt.reshape(NVOX)


# speedup vs baseline: 23.4450x; 1.0081x over previous
import functools

import jax
import jax.numpy as jnp
from jax import lax
from jax.experimental import pallas as pl
from jax.experimental.pallas import tpu as pltpu
from jax.experimental.pallas import tpu_sc as plsc

N_VIEWS = 9
C = 80
H = 60
W = 80
NVOX = 48 * 48 * 48
NW = 32
SLAB = NVOX // NW
NCH = SLAB // 128
VOXEL_SIZE = 0.04
OFFS = (0, 1, W, W + 1)


def _stage0_body(pix_ref, idx_ref, wts_ref, cnt_ref, inv_ref, imz_ref):
    cnt = jnp.zeros((NW, 128), jnp.float32)
    zsum = jnp.zeros((NW, 128), jnp.float32)
    for v in range(N_VIEWS):
        im_x = pix_ref[v, 0, 0]
        im_y = pix_ref[v, 1, 0]
        q2 = pix_ref[v, 2, 0]
        gx = 2.0 * im_x / (W - 1) - 1.0
        gy = 2.0 * im_y / (H - 1) - 1.0
        mask = (jnp.abs(gx) <= 1.0) & (jnp.abs(gy) <= 1.0) & (q2 > 0)
        ix = (gx + 1.0) * 0.5 * (W - 1)
        iy = (gy + 1.0) * 0.5 * (H - 1)
        x0 = jnp.clip(jnp.floor(ix), 0.0, W - 2.0)
        y0 = jnp.clip(jnp.floor(iy), 0.0, H - 2.0)
        wx1 = ix - x0
        wx0 = 1.0 - wx1
        wy1 = iy - y0
        wy0 = 1.0 - wy1
        base = jnp.where(mask, y0 * jnp.float32(W) + x0, 0.0).astype(jnp.int32)
        for ci, off in enumerate(OFFS):
            idx_ref[0, :, v, ci, :] = base + off
        wts_ref[0, :, v, 0, :] = jnp.where(mask, wx0 * wy0, 0.0)
        wts_ref[0, :, v, 1, :] = jnp.where(mask, wx1 * wy0, 0.0)
        wts_ref[0, :, v, 2, :] = jnp.where(mask, wx0 * wy1, 0.0)
        wts_ref[0, :, v, 3, :] = jnp.where(mask, wx1 * wy1, 0.0)
        cnt = cnt + mask.astype(jnp.float32)
        zsum = zsum + jnp.where(mask, q2, 0.0)

    msafe = jnp.where(cnt == 0.0, 1.0, cnt)
    cnt_ref[0, :, 0, :] = cnt
    inv_ref[0, :, 0, :] = 1.0 / msafe
    imz_ref[0, :, 0, :] = zsum / msafe


def _stage0(pix):
    return pl.pallas_call(
        _stage0_body,
        grid=(NCH,),
        in_specs=[
            pl.BlockSpec((N_VIEWS, 3, 1, NW, 128), lambda r: (0, 0, r, 0, 0)),
        ],
        out_specs=[
            pl.BlockSpec((1, NW, N_VIEWS, 4, 128), lambda r: (r, 0, 0, 0, 0)),
            pl.BlockSpec((1, NW, N_VIEWS, 4, 128), lambda r: (r, 0, 0, 0, 0)),
            pl.BlockSpec((1, NW, 1, 128), lambda r: (r, 0, 0, 0)),
            pl.BlockSpec((1, NW, 1, 128), lambda r: (r, 0, 0, 0)),
            pl.BlockSpec((1, NW, 1, 128), lambda r: (r, 0, 0, 0)),
        ],
        out_shape=[
            jax.ShapeDtypeStruct((NCH, NW, N_VIEWS, 4, 128), jnp.int32),
            jax.ShapeDtypeStruct((NCH, NW, N_VIEWS, 4, 128), jnp.float32),
            jax.ShapeDtypeStruct((NCH, NW, 1, 128), jnp.float32),
            jax.ShapeDtypeStruct((NCH, NW, 1, 128), jnp.float32),
            jax.ShapeDtypeStruct((NCH, NW, 1, 128), jnp.float32),
        ],
    )(pix)


def _stagez_body(imz_ref, zn_ref):
    x = imz_ref[...].reshape(NCH * NW, 128)
    pos = x > 0.0
    npos = jnp.sum(pos.astype(jnp.float32))
    mean = jnp.where(npos > 0.0,
                     jnp.sum(jnp.where(pos, x, 0.0)) / jnp.maximum(npos, 1.0),
                     0.0)
    std = jnp.sqrt(jnp.sum(jnp.where(pos, (x - mean) ** 2, 0.0))) + 1e-5
    zn = jnp.where(pos, (x - mean) / std, 0.0)
    zn_ref[...] = zn.reshape(NCH, NW, 1, 128)


def _stagez(imz):
    return pl.pallas_call(
        _stagez_body,
        out_shape=jax.ShapeDtypeStruct((NCH, NW, 1, 128), jnp.float32),
    )(imz)


def _stage1_body(tables, idx_hbm, wts_hbm, inv_hbm, zn_hbm, vol_hbm,
                 idx_s, w_s, inv_s, zn_s, rows, acc, sem_s, sem_a, sem_b):
    wid = lax.axis_index("s") * 2 + lax.axis_index("c")

    def fetch_slabs(r, p):
        pltpu.async_copy(idx_hbm.at[r, wid], idx_s.at[p], sem_s)
        pltpu.async_copy(wts_hbm.at[r, wid], w_s.at[p, :, :, pl.ds(0, 128)],
                         sem_s)
        pltpu.async_copy(inv_hbm.at[r, wid, 0], inv_s.at[p, pl.ds(0, 128)],
                         sem_s)
        pltpu.async_copy(zn_hbm.at[r, wid, 0], zn_s.at[p, pl.ds(0, 128)],
                         sem_s)

    def wait_slabs(p):
        pltpu.make_async_copy(idx_hbm.at[0, 0], idx_s.at[p], sem_s).wait()
        pltpu.make_async_copy(wts_hbm.at[0, 0],
                              w_s.at[p, :, :, pl.ds(0, 128)], sem_s).wait()
        pltpu.make_async_copy(inv_hbm.at[0, 0, 0],
                              inv_s.at[p, pl.ds(0, 128)], sem_s).wait()
        pltpu.make_async_copy(zn_hbm.at[0, 0, 0],
                              zn_s.at[p, pl.ds(0, 128)], sem_s).wait()

    fetch_slabs(0, 0)

    def chunk_body(r, _):
        p = lax.rem(r, 2)
        wait_slabs(p)

        def fire(v, sem):
            return [pltpu.async_copy(tables.at[v].at[idx_s.at[p, v, 2 * half]],
                                     rows.at[v % 2, half], sem)
                    for half in range(2)]

        h = fire(0, sem_a)
        fetch_slabs(jnp.minimum(r + 1, NCH - 1), 1 - p)

        for v in range(N_VIEWS):
            hn = fire(v + 1, (sem_a, sem_b)[(v + 1) % 2]) if v < N_VIEWS - 1 else None
            for hh in h:
                hh.wait()

            def group_body(g, _, v=v):
                w16 = [w_s[p, v, ci, pl.ds(g * 8, 16)] for ci in range(4)]
                for jj in range(8):
                    j = g * 8 + jj
                    w0 = w16[0][jj]
                    w1 = w16[1][jj]
                    w2 = w16[2][jj]
                    w3 = w16[3][jj]
                    for k in range(5):
                        sl0 = pl.ds(16 * k, 16)
                        sl1 = pl.ds(C + 16 * k, 16)
                        s = (w0 * rows[v % 2, 0, j, sl0]
                             + w1 * rows[v % 2, 0, j, sl1]
                             + w2 * rows[v % 2, 1, j, sl0]
                             + w3 * rows[v % 2, 1, j, sl1])
                        if v == 0:
                            acc[j, sl0] = s
                        else:
                            acc[j, sl0] = acc[j, sl0] + s
                return 0

            lax.fori_loop(0, 16, group_body, 0)
            h = hn

        col80 = jnp.full((16,), C, jnp.int32)

        def scale_body(g, _):
            iv16 = inv_s[p, pl.ds(g * 8, 16)]
            for jj in range(8):
                j = g * 8 + jj
                iv = iv16[jj]
                for k in range(5):
                    sl = pl.ds(16 * k, 16)
                    acc[j, sl] = acc[j, sl] * iv
            return 0

        lax.fori_loop(0, 16, scale_body, 0)

        def zcol_body(g, _):
            z16 = zn_s[p, pl.ds(g * 16, 16)]
            rows16 = g * 16 + lax.iota(jnp.int32, 16)
            plsc.store_scatter(acc, [rows16, col80], z16)
            return 0

        lax.fori_loop(0, 8, zcol_body, 0)
        pltpu.sync_copy(acc, vol_hbm.at[pl.ds((r * NW + wid) * 128, 128)])
        return 0

    lax.fori_loop(0, NCH, chunk_body, 0)
    wait_slabs(NCH % 2)


def _stage1(tables, idx4, wts, inv, zn):
    mesh = plsc.VectorSubcoreMesh(core_axis_name="c", subcore_axis_name="s")
    f = functools.partial(
        pl.kernel,
        out_type=jax.ShapeDtypeStruct((NVOX, C + 1), jnp.float32),
        mesh=mesh,
        compiler_params=pltpu.CompilerParams(use_tc_tiling_on_sc=False,
                                             needs_layout_passes=False),
        scratch_types=[
            pltpu.VMEM((2, N_VIEWS, 4, 128), jnp.int32),
            pltpu.VMEM((2, N_VIEWS, 4, 144), jnp.float32),
            pltpu.VMEM((2, 144), jnp.float32),
            pltpu.VMEM((2, 144), jnp.float32),
            pltpu.VMEM((2, 2, 128, 2 * C), jnp.float32),
            pltpu.VMEM((128, C + 1), jnp.float32),
            pltpu.SemaphoreType.DMA,
            pltpu.SemaphoreType.DMA,
            pltpu.SemaphoreType.DMA,
        ],
    )(_stage1_body)
    return f(tables, idx4, wts, inv, zn)


def kernel(feats, proj_matrices, vol_origin_partial):
    tbl = jnp.transpose(feats[:, 0], (0, 2, 3, 1)).reshape(N_VIEWS, H * W, C)
    tbl_next = jnp.pad(tbl[:, 1:], ((0, 0), (0, 1), (0, 0)))
    tables = jnp.concatenate([tbl, tbl_next], axis=2)
    interval = 4
    rngs = [jnp.arange(0, 48 * interval, interval, dtype=jnp.float32)
            for _ in range(3)]
    g = jnp.stack(jnp.meshgrid(rngs[0], rngs[1], rngs[2], indexing='ij'))
    coords_xyz = g.reshape(3, -1).T
    grid = coords_xyz * VOXEL_SIZE + vol_origin_partial[0][None, :]
    rs = jnp.concatenate([grid.T, jnp.ones((1, NVOX), grid.dtype)], axis=0)
    im_p = lax.map(lambda P: P @ rs, proj_matrices[:, 0])
    im_z = im_p[:, 2]
    im_x = im_p[:, 0] / (im_z + 1e-8)
    im_y = im_p[:, 1] / (im_z + 1e-8)
    pix = jnp.stack([im_x, im_y, im_z], axis=1)
    pix = pix.reshape(N_VIEWS, 3, NCH, NW, 128)

    idx4, wts, cnt, inv, imz = _stage0(pix)
    zn = _stagez(imz)
    volume = _stage1(tables, idx4, wts, inv, zn)
    return volume, cnt.reshape(NVOX)
